# Initial kernel scaffold; baseline (speedup 1.0000x reference)
import jax
import jax.numpy as jnp
from jax.experimental import pallas as pl


def kernel(x, edge_index, batch, W1, b1, W2, b2, W3, b3):
    raise NotImplementedError("write your pallas kernel here")


# speedup vs baseline: 60.6530x; 60.6530x over previous
import functools

import jax
import jax.numpy as jnp
from jax import lax
from jax.experimental import pallas as pl
from jax.experimental.pallas import tpu as pltpu
from jax.experimental.pallas import tpu_sc as plsc

_N = 50000
_E = 1600000
_G = 128
_H = 16

_NC = 2
_NS = 16
_NW = _NC * _NS

_NP = 50048
_RPT = _NP // _NS
_NPAD_ROWS = _NP - _N

_EP = 1638400
_ECH = _EP // 128
_CPW = _ECH // _NW
_CB = 16
_CO = _CPW // _CB

_RB = 3128


def _mesh():
    return plsc.VectorSubcoreMesh(core_axis_name="c", subcore_axis_name="s")


_SC_PARAMS = pltpu.CompilerParams(use_tc_tiling_on_sc=False)


def _deg_body(dstz, out, didx, ones_v, zbuf, vout, dacc, sem):
    c = lax.axis_index("c")
    s = lax.axis_index("s")
    wid = s * _NC + c
    rbase = s * _RPT

    def fill1(i, carry):
        zbuf[pl.ds(i * 16, 16)] = jnp.zeros((16,), jnp.float32)
        return carry

    lax.fori_loop(0, (_RPT + 15) // 16, fill1, 0)
    for k in range(8):
        ones_v[pl.ds(k * 16, 16)] = jnp.ones((16,), jnp.float32)
    pltpu.sync_copy(zbuf.at[pl.ds(0, _RPT)], dacc.at[pl.ds(rbase, _RPT)])
    plsc.subcore_barrier()
    cbase = wid * _CPW

    def outer(g, carry):
        ch = cbase + g * _CB
        pltpu.sync_copy(dstz.at[pl.ds(ch, _CB)], didx)
        for j in range(_CB):
            pltpu.async_copy(ones_v, dacc.at[didx.at[j]], sem, add=True)
        for j in range(_CB):
            pltpu.make_async_copy(ones_v, dacc.at[didx.at[j]], sem).wait()
        return carry

    lax.fori_loop(0, _CO, outer, 0)
    plsc.subcore_barrier()
    pltpu.sync_copy(dacc.at[pl.ds(rbase, _RPT)], vout)
    pltpu.sync_copy(vout, out.at[pl.ds(c * _NP + rbase, _RPT)])


def _deg_call(dstz):
    return pl.kernel(
        _deg_body,
        out_type=jax.ShapeDtypeStruct((_NC * _NP,), jnp.float32),
        mesh=_mesh(),
        scratch_types=[
            pltpu.VMEM((_CB, 128), jnp.int32),
            pltpu.VMEM((128,), jnp.float32),
            pltpu.VMEM((16 * ((_RPT + 15) // 16),), jnp.float32),
            pltpu.VMEM((_RPT,), jnp.float32),
            pltpu.VMEM_SHARED((_NP,), jnp.float32),
            pltpu.SemaphoreType.DMA,
        ],
        compiler_params=_SC_PARAMS,
    )(dstz)


_ZR = _RPT // 8


def _mp_body(y, srcz, dstz, out, sidx, didx, rows, zbuf, acc,
             sem_g, sem_s):
    c = lax.axis_index("c")
    s = lax.axis_index("s")
    wid = s * _NC + c
    rbase = s * _RPT

    def fill(i, carry):
        zbuf[i] = jnp.zeros((_H,), jnp.float32)
        return carry

    lax.fori_loop(0, _ZR, fill, 0)
    for k in range(8):
        pltpu.sync_copy(zbuf, acc.at[pl.ds(rbase + k * _ZR, _ZR)])
    plsc.subcore_barrier()
    cbase = wid * _CPW

    def outer(g, carry):
        ch = cbase + g * _CB
        pltpu.sync_copy(srcz.at[pl.ds(ch, _CB)], sidx)
        pltpu.sync_copy(dstz.at[pl.ds(ch, _CB)], didx)
        for j in range(_CB):
            pltpu.async_copy(y.at[sidx.at[j]], rows.at[j], sem_g)
        for j in range(_CB):
            pltpu.make_async_copy(y.at[sidx.at[j]], rows.at[j], sem_g).wait()
        for j in range(_CB):
            pltpu.async_copy(rows.at[j], acc.at[didx.at[j]], sem_s, add=True)
        for j in range(_CB):
            pltpu.make_async_copy(rows.at[j], acc.at[didx.at[j]], sem_s).wait()
        return carry

    lax.fori_loop(0, _CO, outer, 0)
    plsc.subcore_barrier()
    obase = c * _NP + rbase
    for k in range(8):
        pltpu.sync_copy(acc.at[pl.ds(rbase + k * _ZR, _ZR)], zbuf)
        pltpu.sync_copy(zbuf, out.at[pl.ds(obase + k * _ZR, _ZR)])


@functools.cache
def _mp_kernel():
    return pl.kernel(
        _mp_body,
        out_type=jax.ShapeDtypeStruct((_NC * _NP, _H), jnp.float32),
        mesh=_mesh(),
        scratch_types=[
            pltpu.VMEM((_CB, 128), jnp.int32),
            pltpu.VMEM((_CB, 128), jnp.int32),
            pltpu.VMEM((_CB, 128, _H), jnp.float32),
            pltpu.VMEM((_ZR, _H), jnp.float32),
            pltpu.VMEM_SHARED((_NP, _H), jnp.float32),
            pltpu.SemaphoreType.DMA,
            pltpu.SemaphoreType.DMA,
        ],
        compiler_params=_SC_PARAMS,
    )


def _mp_call(y, srcz, dstz):
    return _mp_kernel()(y, srcz, dstz)


def _xw_body(x_ref, w_ref, o_ref):
    o_ref[...] = jnp.dot(x_ref[...], w_ref[...],
                         preferred_element_type=jnp.float32)


def _xw_call(x, w):
    f_in = x.shape[1]
    f_out = w.shape[1]
    return pl.pallas_call(
        _xw_body,
        grid=(_NP // _RB,),
        in_specs=[
            pl.BlockSpec((_RB, f_in), lambda i: (i, 0)),
            pl.BlockSpec((f_in, f_out), lambda i: (0, 0)),
        ],
        out_specs=pl.BlockSpec((_RB, f_out), lambda i: (i, 0)),
        out_shape=jax.ShapeDtypeStruct((_NP, f_out), jnp.float32),
    )(x, w)


def _y1_body(xw_ref, d0_ref, d1_ref, y_ref, dis_ref):
    deg = d0_ref[...] + d1_ref[...] + 1.0
    dis = lax.rsqrt(deg)
    dis_ref[...] = dis
    y_ref[...] = xw_ref[...] * dis


def _y1_call(xw, d0, d1):
    return pl.pallas_call(
        _y1_body,
        grid=(_NP // _RB,),
        in_specs=[
            pl.BlockSpec((_RB, _H), lambda i: (i, 0)),
            pl.BlockSpec((_RB, 1), lambda i: (i, 0)),
            pl.BlockSpec((_RB, 1), lambda i: (i, 0)),
        ],
        out_specs=[
            pl.BlockSpec((_RB, _H), lambda i: (i, 0)),
            pl.BlockSpec((_RB, 1), lambda i: (i, 0)),
        ],
        out_shape=[
            jax.ShapeDtypeStruct((_NP, _H), jnp.float32),
            jax.ShapeDtypeStruct((_NP, 1), jnp.float32),
        ],
    )(xw, d0, d1)


def _mid_body(a0_ref, a1_ref, y1_ref, dis_ref, w_ref, b_ref, y2_ref):
    dis = dis_ref[...]
    h = jax.nn.relu((a0_ref[...] + a1_ref[...] + y1_ref[...]) * dis
                    + b_ref[...])
    y2_ref[...] = jnp.dot(h, w_ref[...],
                          preferred_element_type=jnp.float32) * dis


def _mid_call(a0, a1, y1, dis, w2, b1):
    return pl.pallas_call(
        _mid_body,
        grid=(_NP // _RB,),
        in_specs=[
            pl.BlockSpec((_RB, _H), lambda i: (i, 0)),
            pl.BlockSpec((_RB, _H), lambda i: (i, 0)),
            pl.BlockSpec((_RB, _H), lambda i: (i, 0)),
            pl.BlockSpec((_RB, 1), lambda i: (i, 0)),
            pl.BlockSpec((_H, _H), lambda i: (0, 0)),
            pl.BlockSpec((1, _H), lambda i: (0, 0)),
        ],
        out_specs=pl.BlockSpec((_RB, _H), lambda i: (i, 0)),
        out_shape=jax.ShapeDtypeStruct((_NP, _H), jnp.float32),
    )(a0, a1, y1, dis, w2, b1)


def _fin_body(a0_ref, a1_ref, y2_ref, dis_ref, b2_ref, w3_ref, batch_ref,
              b3_ref, out_ref, s_ref):
    i = pl.program_id(0)
    nblk = pl.num_programs(0)

    @pl.when(i == 0)
    def _():
        s_ref[...] = jnp.zeros_like(s_ref)

    dis = dis_ref[...]
    h2 = jax.nn.relu((a0_ref[...] + a1_ref[...] + y2_ref[...]) * dis
                     + b2_ref[...])
    z = jnp.dot(h2, w3_ref[...], preferred_element_type=jnp.float32)
    zz = jnp.concatenate([z, jnp.ones((_RB, 1), jnp.float32)], axis=1)
    ids = lax.broadcasted_iota(jnp.int32, (_RB, _G), 1)
    oh = (batch_ref[...] == ids).astype(jnp.float32)
    s_ref[...] += lax.dot_general(oh, zz, (((0,), (0,)), ((), ())),
                                  preferred_element_type=jnp.float32)

    @pl.when(i == nblk - 1)
    def _():
        s = s_ref[...]
        out_ref[...] = s[:, 0:2] / jnp.maximum(s[:, 2:3], 1.0) + b3_ref[...]


def _fin_call(a0, a1, y2, dis, b2, w3, batch2, b3):
    return pl.pallas_call(
        _fin_body,
        grid=(_NP // _RB,),
        in_specs=[
            pl.BlockSpec((_RB, _H), lambda i: (i, 0)),
            pl.BlockSpec((_RB, _H), lambda i: (i, 0)),
            pl.BlockSpec((_RB, _H), lambda i: (i, 0)),
            pl.BlockSpec((_RB, 1), lambda i: (i, 0)),
            pl.BlockSpec((1, _H), lambda i: (0, 0)),
            pl.BlockSpec((_H, 2), lambda i: (0, 0)),
            pl.BlockSpec((_RB, 1), lambda i: (i, 0)),
            pl.BlockSpec((1, 2), lambda i: (0, 0)),
        ],
        out_specs=pl.BlockSpec((_G, 2), lambda i: (0, 0)),
        out_shape=jax.ShapeDtypeStruct((_G, 2), jnp.float32),
        scratch_shapes=[pltpu.VMEM((_G, 3), jnp.float32)],
    )(a0, a1, y2, dis, b2, w3, batch2, b3)


def kernel(x, edge_index, batch, W1, b1, W2, b2, W3, b3):
    src = edge_index[0]
    dst = edge_index[1]
    pad_idx = _N + (jnp.arange(_EP - _E, dtype=jnp.int32) % _NPAD_ROWS)
    srcz = jnp.concatenate([src, pad_idx]).reshape(_ECH, 128)
    dstz = jnp.concatenate([dst, pad_idx]).reshape(_ECH, 128)

    x_p = jnp.pad(x, ((0, _NP - _N), (0, 0)))
    batch_p = jnp.pad(batch, (0, _NP - _N), constant_values=_G)
    batch2 = batch_p.reshape(_NP, 1)

    b1r = b1.reshape(1, _H)
    b2r = b2.reshape(1, _H)
    b3r = b3.reshape(1, 2)

    xw1 = _xw_call(x_p, W1)
    degp = _deg_call(dstz)
    d0 = degp[:_NP].reshape(_NP, 1)
    d1 = degp[_NP:].reshape(_NP, 1)

    y1, dis = _y1_call(xw1, d0, d1)

    acc1 = _mp_call(y1, srcz, dstz)
    y2 = _mid_call(acc1[:_NP], acc1[_NP:], y1, dis, W2, b1r)

    acc2 = _mp_call(y2, srcz, dstz)
    return _fin_call(acc2[:_NP], acc2[_NP:], y2, dis, b2r, W3, batch2, b3r)


# speedup vs baseline: 68.5233x; 1.1298x over previous
import functools

import jax
import jax.numpy as jnp
from jax import lax
from jax.experimental import pallas as pl
from jax.experimental.pallas import tpu as pltpu
from jax.experimental.pallas import tpu_sc as plsc

_N = 50000
_E = 1600000
_G = 128
_H = 16

_NC = 2
_NS = 16
_NW = _NC * _NS

_NP = 65536
_RPT = _NP // _NS
_NPAD_ROWS = _NP - _N

_EP = 1638400
_ECH = _EP // 128
_CPW = _ECH // _NW
_CB = 16
_CO = _CPW // _CB

_RB = 4096


def _mesh():
    return plsc.VectorSubcoreMesh(core_axis_name="c", subcore_axis_name="s")


_SC_PARAMS = pltpu.CompilerParams(use_tc_tiling_on_sc=False)


def _deg_body(dstz, out, didx, ones_v, zbuf, vout, dacc, sem):
    c = lax.axis_index("c")
    s = lax.axis_index("s")
    wid = s * _NC + c
    rbase = s * _RPT

    def fill1(i, carry):
        zbuf[pl.ds(i * 16, 16)] = jnp.zeros((16,), jnp.float32)
        return carry

    lax.fori_loop(0, (_RPT + 15) // 16, fill1, 0)
    for k in range(8):
        ones_v[pl.ds(k * 16, 16)] = jnp.ones((16,), jnp.float32)
    pltpu.sync_copy(zbuf.at[pl.ds(0, _RPT)], dacc.at[pl.ds(rbase, _RPT)])
    plsc.subcore_barrier()
    cbase = wid * _CPW

    def outer(g, carry):
        ch = cbase + g * _CB
        pltpu.sync_copy(dstz.at[pl.ds(ch, _CB)], didx)
        for j in range(_CB):
            pltpu.async_copy(ones_v, dacc.at[didx.at[j]], sem, add=True)
        for j in range(_CB):
            pltpu.make_async_copy(ones_v, dacc.at[didx.at[j]], sem).wait()
        return carry

    lax.fori_loop(0, _CO, outer, 0)
    plsc.subcore_barrier()
    pltpu.sync_copy(dacc.at[pl.ds(rbase, _RPT)], vout)
    pltpu.sync_copy(vout, out.at[pl.ds(c * _NP + rbase, _RPT)])


def _deg_call(dstz):
    return pl.kernel(
        _deg_body,
        out_type=jax.ShapeDtypeStruct((_NC * _NP,), jnp.float32),
        mesh=_mesh(),
        scratch_types=[
            pltpu.VMEM((_CB, 128), jnp.int32),
            pltpu.VMEM((128,), jnp.float32),
            pltpu.VMEM((16 * ((_RPT + 15) // 16),), jnp.float32),
            pltpu.VMEM((_RPT,), jnp.float32),
            pltpu.VMEM_SHARED((_NP,), jnp.float32),
            pltpu.SemaphoreType.DMA,
        ],
        compiler_params=_SC_PARAMS,
    )(dstz)


_ZR = _RPT // 8


def _mp_body(y, srcz, dstz, out, sidx, didx, rows, zbuf, acc,
             sem_g, sem_s):
    c = lax.axis_index("c")
    s = lax.axis_index("s")
    wid = s * _NC + c
    rbase = s * _RPT

    def fill(i, carry):
        zbuf[i] = jnp.zeros((_H,), jnp.float32)
        return carry

    lax.fori_loop(0, _ZR, fill, 0)
    for k in range(8):
        pltpu.sync_copy(zbuf, acc.at[pl.ds(rbase + k * _ZR, _ZR)])
    plsc.subcore_barrier()
    cbase = wid * _CPW

    def outer(g, carry):
        ch = cbase + g * _CB
        pltpu.sync_copy(srcz.at[pl.ds(ch, _CB)], sidx)
        pltpu.sync_copy(dstz.at[pl.ds(ch, _CB)], didx)
        for j in range(_CB):
            pltpu.async_copy(y.at[sidx.at[j]], rows.at[j], sem_g)
        for j in range(_CB):
            pltpu.make_async_copy(y.at[sidx.at[j]], rows.at[j], sem_g).wait()
        for j in range(_CB):
            pltpu.async_copy(rows.at[j], acc.at[didx.at[j]], sem_s, add=True)
        for j in range(_CB):
            pltpu.make_async_copy(rows.at[j], acc.at[didx.at[j]], sem_s).wait()
        return carry

    lax.fori_loop(0, _CO, outer, 0)
    plsc.subcore_barrier()
    obase = c * _NP + rbase
    for k in range(8):
        pltpu.sync_copy(acc.at[pl.ds(rbase + k * _ZR, _ZR)], zbuf)
        pltpu.sync_copy(zbuf, out.at[pl.ds(obase + k * _ZR, _ZR)])


@functools.cache
def _mp_kernel():
    return pl.kernel(
        _mp_body,
        out_type=jax.ShapeDtypeStruct((_NC * _NP, _H), jnp.float32),
        mesh=_mesh(),
        scratch_types=[
            pltpu.VMEM((_CB, 128), jnp.int32),
            pltpu.VMEM((_CB, 128), jnp.int32),
            pltpu.VMEM((_CB, 128, _H), jnp.float32),
            pltpu.VMEM((_ZR, _H), jnp.float32),
            pltpu.VMEM_SHARED((_NP, _H), jnp.float32),
            pltpu.SemaphoreType.DMA,
            pltpu.SemaphoreType.DMA,
        ],
        compiler_params=_SC_PARAMS,
    )


def _mp_call(y, srcz, dstz):
    return _mp_kernel()(y, srcz, dstz)


def _xw_body(x_ref, w_ref, o_ref):
    o_ref[...] = jnp.dot(x_ref[...], w_ref[...],
                         preferred_element_type=jnp.float32)


def _xw_call(x, w):
    f_in = x.shape[1]
    f_out = w.shape[1]
    return pl.pallas_call(
        _xw_body,
        grid=(_NP // _RB,),
        in_specs=[
            pl.BlockSpec((_RB, f_in), lambda i: (i, 0)),
            pl.BlockSpec((f_in, f_out), lambda i: (0, 0)),
        ],
        out_specs=pl.BlockSpec((_RB, f_out), lambda i: (i, 0)),
        out_shape=jax.ShapeDtypeStruct((_NP, f_out), jnp.float32),
    )(x, w)


def _y1_body(xw_ref, d0_ref, d1_ref, y_ref, dis_ref):
    deg = d0_ref[...] + d1_ref[...] + 1.0
    dis = lax.rsqrt(deg)[:, None]
    disb = jnp.broadcast_to(dis, (_RB, _H))
    dis_ref[...] = disb
    y_ref[...] = xw_ref[...] * disb


def _y1_call(xw, degp):
    nb = _NP // _RB
    return pl.pallas_call(
        _y1_body,
        grid=(nb,),
        in_specs=[
            pl.BlockSpec((_RB, _H), lambda i: (i, 0)),
            pl.BlockSpec((_RB,), lambda i: (i,)),
            pl.BlockSpec((_RB,), lambda i: (i + _NP // _RB,)),
        ],
        out_specs=[
            pl.BlockSpec((_RB, _H), lambda i: (i, 0)),
            pl.BlockSpec((_RB, _H), lambda i: (i, 0)),
        ],
        out_shape=[
            jax.ShapeDtypeStruct((_NP, _H), jnp.float32),
            jax.ShapeDtypeStruct((_NP, _H), jnp.float32),
        ],
    )(xw, degp, degp)


def _mid_body(a0_ref, a1_ref, y1_ref, dis_ref, w_ref, b_ref, y2_ref):
    dis = dis_ref[...]
    h = jax.nn.relu((a0_ref[...] + a1_ref[...] + y1_ref[...]) * dis
                    + b_ref[...])
    y2_ref[...] = jnp.dot(h, w_ref[...],
                          preferred_element_type=jnp.float32) * dis


def _mid_call(accp, y1, dis, w2, b1):
    return pl.pallas_call(
        _mid_body,
        grid=(_NP // _RB,),
        in_specs=[
            pl.BlockSpec((_RB, _H), lambda i: (i, 0)),
            pl.BlockSpec((_RB, _H), lambda i: (i + _NP // _RB, 0)),
            pl.BlockSpec((_RB, _H), lambda i: (i, 0)),
            pl.BlockSpec((_RB, _H), lambda i: (i, 0)),
            pl.BlockSpec((_H, _H), lambda i: (0, 0)),
            pl.BlockSpec((1, _H), lambda i: (0, 0)),
        ],
        out_specs=pl.BlockSpec((_RB, _H), lambda i: (i, 0)),
        out_shape=jax.ShapeDtypeStruct((_NP, _H), jnp.float32),
    )(accp, accp, y1, dis, w2, b1)


def _fin_body(a0_ref, a1_ref, y2_ref, dis_ref, b2_ref, w3_ref, batch_ref,
              b3_ref, out_ref, s_ref):
    i = pl.program_id(0)
    nblk = pl.num_programs(0)

    @pl.when(i == 0)
    def _():
        s_ref[...] = jnp.zeros_like(s_ref)

    dis = dis_ref[...]
    h2 = jax.nn.relu((a0_ref[...] + a1_ref[...] + y2_ref[...]) * dis
                     + b2_ref[...])
    z = jnp.dot(h2, w3_ref[...], preferred_element_type=jnp.float32)
    zz = jnp.concatenate([z, jnp.ones((_RB, 1), jnp.float32)], axis=1)
    ids = lax.broadcasted_iota(jnp.int32, (_RB, _G), 1)
    oh = (batch_ref[...][:, None] == ids).astype(jnp.float32)
    s_ref[...] += lax.dot_general(oh, zz, (((0,), (0,)), ((), ())),
                                  preferred_element_type=jnp.float32)

    @pl.when(i == nblk - 1)
    def _():
        s = s_ref[...]
        out_ref[...] = s[:, 0:2] / jnp.maximum(s[:, 2:3], 1.0) + b3_ref[...]


def _fin_call(accp, y2, dis, b2, w3, batch_p, b3):
    return pl.pallas_call(
        _fin_body,
        grid=(_NP // _RB,),
        in_specs=[
            pl.BlockSpec((_RB, _H), lambda i: (i, 0)),
            pl.BlockSpec((_RB, _H), lambda i: (i + _NP // _RB, 0)),
            pl.BlockSpec((_RB, _H), lambda i: (i, 0)),
            pl.BlockSpec((_RB, _H), lambda i: (i, 0)),
            pl.BlockSpec((1, _H), lambda i: (0, 0)),
            pl.BlockSpec((_H, 2), lambda i: (0, 0)),
            pl.BlockSpec((_RB,), lambda i: (i,)),
            pl.BlockSpec((1, 2), lambda i: (0, 0)),
        ],
        out_specs=pl.BlockSpec((_G, 2), lambda i: (0, 0)),
        out_shape=jax.ShapeDtypeStruct((_G, 2), jnp.float32),
        scratch_shapes=[pltpu.VMEM((_G, 3), jnp.float32)],
    )(accp, accp, y2, dis, b2, w3, batch_p, b3)


def kernel(x, edge_index, batch, W1, b1, W2, b2, W3, b3):
    src = edge_index[0]
    dst = edge_index[1]
    pad_idx = _N + (jnp.arange(_EP - _E, dtype=jnp.int32) % _NPAD_ROWS)
    srcz = jnp.concatenate([src, pad_idx]).reshape(_ECH, 128)
    dstz = jnp.concatenate([dst, pad_idx]).reshape(_ECH, 128)

    x_p = jnp.pad(x, ((0, _NP - _N), (0, 0)))
    batch_p = jnp.pad(batch, (0, _NP - _N), constant_values=_G)

    b1r = b1.reshape(1, _H)
    b2r = b2.reshape(1, _H)
    b3r = b3.reshape(1, 2)

    xw1 = _xw_call(x_p, W1)
    degp = _deg_call(dstz)

    y1, dis = _y1_call(xw1, degp)

    acc1 = _mp_call(y1, srcz, dstz)
    y2 = _mid_call(acc1, y1, dis, W2, b1r)

    acc2 = _mp_call(y2, srcz, dstz)
    return _fin_call(acc2, y2, dis, b2r, W3, batch_p, b3r)
